# baseline (speedup 1.0000x reference)
import functools

import numpy as np

import jax
import jax.numpy as jnp
from jax import lax
from jax.experimental import pallas as pl
from jax.experimental.pallas import tpu as pltpu
from jax.experimental.pallas import tpu_sc as plsc

_B, _T = 16384, 200
_EMB = 64
_HID = 200
_NC, _NS = 2, 16
_NW = _NC * _NS
_BAGS = _B // _NW
_SPLIT = 104
_RING = 4
_UNROLL = 8
_VOCAB = 100000

_PERM = np.concatenate([
    np.arange(0, 32, 2), np.arange(1, 32, 2),
    np.arange(32, 64, 2), np.arange(33, 64, 2)])


def _sc_embed_mean(x, table, inv_den):
    mesh = plsc.VectorSubcoreMesh(core_axis_name="c", subcore_axis_name="s")

    @functools.partial(
        pl.kernel,
        mesh=mesh,
        out_type=jax.ShapeDtypeStruct((_B // 2, 2 * _EMB), jnp.float32),
        scratch_types=[
            pltpu.VMEM((_RING, _T), jnp.int32),
            pltpu.VMEM((_RING, _T, _EMB // 2), jnp.int32),
            pltpu.VMEM((_RING, _EMB), jnp.float32),
            pltpu.VMEM((_BAGS,), jnp.float32),
            pltpu.SemaphoreType.DMA,
            pltpu.SemaphoreType.DMA,
            pltpu.SemaphoreType.DMA,
        ],
        compiler_params=pltpu.CompilerParams(use_tc_tiling_on_sc=False,
                                             needs_layout_passes=False),
    )
    def body(x_hbm, tab_hbm, den_hbm, out_hbm, idx_v, rows_v, acc_v, den_v,
             isem, rsem, ssem):
        wid = lax.axis_index("s") * _NC + lax.axis_index("c")
        base = wid * _BAGS

        pltpu.sync_copy(den_hbm.at[pl.ds(base, _BAGS)], den_v)

        def fire_idx(bag, slot):
            pltpu.async_copy(x_hbm.at[base + bag], idx_v.at[slot], isem)

        def wait_idx(slot):
            pltpu.make_async_copy(x_hbm.at[base], idx_v.at[slot], isem).wait()

        def fire_rows(slot):
            pltpu.async_copy(
                tab_hbm.at[idx_v.at[slot, pl.ds(0, _SPLIT)]],
                rows_v.at[slot, pl.ds(0, _SPLIT)], rsem)
            pltpu.async_copy(
                tab_hbm.at[idx_v.at[slot, pl.ds(_SPLIT, _T - _SPLIT)]],
                rows_v.at[slot, pl.ds(_SPLIT, _T - _SPLIT)], rsem)

        def wait_rows(slot):
            pltpu.make_async_copy(tab_hbm.at[pl.ds(0, _T)],
                                  rows_v.at[slot], rsem).wait()

        def out_slice(i, half):
            return out_hbm.at[(base + i) // 2, pl.ds(half * _EMB, _EMB)]

        def drain_store(slot):
            pltpu.make_async_copy(acc_v.at[slot],
                                  out_hbm.at[0, pl.ds(0, _EMB)], ssem).wait()

        for s in range(_RING):
            fire_idx(s, s)
        for s in range(2):
            wait_idx(s)
            fire_rows(s)

        def quad_body(j, carry):
            i0 = j * _RING
            for b in range(_RING):
                i = i0 + b
                @pl.when(i + 2 < _BAGS)
                def _():
                    wait_idx((b + 2) % _RING)
                    fire_rows((b + 2) % _RING)

                wait_rows(b)

                @pl.when(i + _RING < _BAGS)
                def _():
                    fire_idx(i + _RING, b)

                zero = jnp.zeros((16,), jnp.float32)

                def red(tt, accs):
                    c0, c1, c2, c3 = accs
                    for r in range(_UNROLL):
                        t = tt * _UNROLL + r
                        r0 = plsc.bitcast(
                            rows_v[b, t, pl.ds(0, 16)], jnp.bfloat16)
                        r1 = plsc.bitcast(
                            rows_v[b, t, pl.ds(16, 16)], jnp.bfloat16)
                        a0, b0 = plsc.unpack(
                            r0, format=plsc.PackFormat.INTERLEAVED,
                            preferred_element_type=jnp.float32)
                        a1, b1 = plsc.unpack(
                            r1, format=plsc.PackFormat.INTERLEAVED,
                            preferred_element_type=jnp.float32)
                        c0 = c0 + a0
                        c1 = c1 + b0
                        c2 = c2 + a1
                        c3 = c3 + b1
                    return (c0, c1, c2, c3)

                c0, c1, c2, c3 = lax.fori_loop(
                    0, _T // _UNROLL, red, (zero, zero, zero, zero))

                d = plsc.load_gather(den_v, [jnp.full((16,), i, jnp.int32)])

                @pl.when(i >= _RING)
                def _():
                    drain_store(b)

                acc_v[b, pl.ds(0, 16)] = c0 * d
                acc_v[b, pl.ds(16, 16)] = c1 * d
                acc_v[b, pl.ds(32, 16)] = c2 * d
                acc_v[b, pl.ds(48, 16)] = c3 * d
                pltpu.async_copy(acc_v.at[b], out_slice(i, b % 2), ssem)
            return carry

        lax.fori_loop(0, _BAGS // _RING, quad_body, 0)

        for s in range(_RING):
            drain_store(s)

    return body(x, table, inv_den)


_BLK = 2048


def _tc_mlp(avg2, W1d, b1d, W2d, b2d):
    def body(a_ref, w1_ref, b1_ref, w2_ref, b2_ref, o_ref):
        h = jnp.maximum(
            jnp.dot(a_ref[...], w1_ref[...], preferred_element_type=jnp.float32)
            + b1_ref[...], 0.0)
        o_ref[...] = (jnp.dot(h, w2_ref[...], preferred_element_type=jnp.float32)
                      + b2_ref[...])

    return pl.pallas_call(
        body,
        grid=(_B // 2 // _BLK,),
        in_specs=[
            pl.BlockSpec((_BLK, 2 * _EMB), lambda i: (i, 0)),
            pl.BlockSpec((2 * _EMB, 2 * _HID), lambda i: (0, 0)),
            pl.BlockSpec((1, 2 * _HID), lambda i: (0, 0)),
            pl.BlockSpec((2 * _HID, 4), lambda i: (0, 0)),
            pl.BlockSpec((1, 4), lambda i: (0, 0)),
        ],
        out_specs=pl.BlockSpec((_BLK, 4), lambda i: (i, 0)),
        out_shape=jax.ShapeDtypeStruct((_B // 2, 4), jnp.float32),
    )(avg2, W1d, b1d, W2d, b2d)


def kernel(X, lengths, emb_table, W1, b1, W2, b2):
    X = X.astype(jnp.int32)
    inv_den = 1.0 / jnp.maximum(lengths.astype(jnp.float32), 1.0)
    table_i32 = jax.lax.bitcast_convert_type(
        emb_table.astype(jnp.bfloat16).reshape(_VOCAB, _EMB // 2, 2),
        jnp.int32)
    avg2 = _sc_embed_mean(X, table_i32, inv_den)

    W1p = W1[_PERM, :]
    zeros1 = jnp.zeros_like(W1)
    W1d = jnp.concatenate([
        jnp.concatenate([W1p, zeros1], axis=1),
        jnp.concatenate([zeros1, W1p], axis=1)], axis=0)
    b1d = jnp.concatenate([b1, b1]).reshape(1, 2 * _HID)
    zeros2 = jnp.zeros_like(W2)
    W2d = jnp.concatenate([
        jnp.concatenate([W2, zeros2], axis=1),
        jnp.concatenate([zeros2, W2], axis=1)], axis=0)
    b2d = jnp.concatenate([b2, b2]).reshape(1, 4)

    out2 = _tc_mlp(avg2, W1d, b1d, W2d, b2d)
    return out2.reshape(_B, 2)


# speedup vs baseline: 1.3487x; 1.3487x over previous
import functools

import numpy as np

import jax
import jax.numpy as jnp
from jax import lax
from jax.experimental import pallas as pl
from jax.experimental.pallas import tpu as pltpu
from jax.experimental.pallas import tpu_sc as plsc

_B, _T = 16384, 200
_EMB = 64
_HID = 200
_NC, _NS = 2, 16
_NW = _NC * _NS
_BAGS = _B // _NW
_SPLIT = 104
_RING = 4
_UNROLL = 8
_VOCAB = 100000

_PERM = np.concatenate([
    np.arange(0, 16), np.arange(32, 48),
    np.arange(16, 32), np.arange(48, 64)])


_PACK_BLK = 4000


def _tc_pack_table(table):
    def body(t_ref, o_ref):
        lo = t_ref[:, :32].astype(jnp.bfloat16)
        hi = t_ref[:, 32:].astype(jnp.bfloat16)
        lo32 = jax.lax.bitcast_convert_type(lo, jnp.uint16).astype(jnp.uint32)
        hi32 = jax.lax.bitcast_convert_type(hi, jnp.uint16).astype(jnp.uint32)
        o_ref[...] = jax.lax.bitcast_convert_type(
            lo32 | (hi32 << 16), jnp.int32)

    return pl.pallas_call(
        body,
        grid=(_VOCAB // _PACK_BLK,),
        in_specs=[pl.BlockSpec((_PACK_BLK, _EMB), lambda i: (i, 0))],
        out_specs=pl.BlockSpec((_PACK_BLK, _EMB // 2), lambda i: (i, 0)),
        out_shape=jax.ShapeDtypeStruct((_VOCAB, _EMB // 2), jnp.int32),
    )(table)


def _sc_embed_mean(x, table, inv_den):
    mesh = plsc.VectorSubcoreMesh(core_axis_name="c", subcore_axis_name="s")

    @functools.partial(
        pl.kernel,
        mesh=mesh,
        out_type=jax.ShapeDtypeStruct((_B // 2, 2 * _EMB), jnp.float32),
        scratch_types=[
            pltpu.VMEM((_RING, _T), jnp.int32),
            pltpu.VMEM((_RING, _T, _EMB // 2), jnp.int32),
            pltpu.VMEM((_RING, _EMB), jnp.float32),
            pltpu.VMEM((_BAGS,), jnp.float32),
            pltpu.SemaphoreType.DMA,
            pltpu.SemaphoreType.DMA,
            pltpu.SemaphoreType.DMA,
        ],
        compiler_params=pltpu.CompilerParams(use_tc_tiling_on_sc=False,
                                             needs_layout_passes=False),
    )
    def body(x_hbm, tab_hbm, den_hbm, out_hbm, idx_v, rows_v, acc_v, den_v,
             isem, rsem, ssem):
        wid = lax.axis_index("s") * _NC + lax.axis_index("c")
        base = wid * _BAGS

        pltpu.sync_copy(den_hbm.at[pl.ds(base, _BAGS)], den_v)

        def fire_idx(bag, slot):
            pltpu.async_copy(x_hbm.at[base + bag], idx_v.at[slot], isem)

        def wait_idx(slot):
            pltpu.make_async_copy(x_hbm.at[base], idx_v.at[slot], isem).wait()

        def fire_rows(slot):
            pltpu.async_copy(
                tab_hbm.at[idx_v.at[slot, pl.ds(0, _SPLIT)]],
                rows_v.at[slot, pl.ds(0, _SPLIT)], rsem)
            pltpu.async_copy(
                tab_hbm.at[idx_v.at[slot, pl.ds(_SPLIT, _T - _SPLIT)]],
                rows_v.at[slot, pl.ds(_SPLIT, _T - _SPLIT)], rsem)

        def wait_rows(slot):
            pltpu.make_async_copy(tab_hbm.at[pl.ds(0, _T)],
                                  rows_v.at[slot], rsem).wait()

        def out_slice(i, half):
            return out_hbm.at[(base + i) // 2, pl.ds(half * _EMB, _EMB)]

        def drain_store(slot):
            pltpu.make_async_copy(acc_v.at[slot],
                                  out_hbm.at[0, pl.ds(0, _EMB)], ssem).wait()

        for s in range(_RING):
            fire_idx(s, s)
        for s in range(2):
            wait_idx(s)
            fire_rows(s)

        def quad_body(j, carry):
            i0 = j * _RING
            for b in range(_RING):
                i = i0 + b
                @pl.when(i + 2 < _BAGS)
                def _():
                    wait_idx((b + 2) % _RING)
                    fire_rows((b + 2) % _RING)

                wait_rows(b)

                @pl.when(i + _RING < _BAGS)
                def _():
                    fire_idx(i + _RING, b)

                zero = jnp.zeros((16,), jnp.float32)

                def red(tt, accs):
                    c0, c1, c2, c3 = accs
                    for r in range(_UNROLL):
                        t = tt * _UNROLL + r
                        r0 = plsc.bitcast(
                            rows_v[b, t, pl.ds(0, 16)], jnp.bfloat16)
                        r1 = plsc.bitcast(
                            rows_v[b, t, pl.ds(16, 16)], jnp.bfloat16)
                        a0, b0 = plsc.unpack(
                            r0, format=plsc.PackFormat.INTERLEAVED,
                            preferred_element_type=jnp.float32)
                        a1, b1 = plsc.unpack(
                            r1, format=plsc.PackFormat.INTERLEAVED,
                            preferred_element_type=jnp.float32)
                        c0 = c0 + a0
                        c1 = c1 + b0
                        c2 = c2 + a1
                        c3 = c3 + b1
                    return (c0, c1, c2, c3)

                c0, c1, c2, c3 = lax.fori_loop(
                    0, _T // _UNROLL, red, (zero, zero, zero, zero))

                d = plsc.load_gather(den_v, [jnp.full((16,), i, jnp.int32)])

                @pl.when(i >= _RING)
                def _():
                    drain_store(b)

                acc_v[b, pl.ds(0, 16)] = c0 * d
                acc_v[b, pl.ds(16, 16)] = c1 * d
                acc_v[b, pl.ds(32, 16)] = c2 * d
                acc_v[b, pl.ds(48, 16)] = c3 * d
                pltpu.async_copy(acc_v.at[b], out_slice(i, b % 2), ssem)
            return carry

        lax.fori_loop(0, _BAGS // _RING, quad_body, 0)

        for s in range(_RING):
            drain_store(s)

    return body(x, table, inv_den)


_BLK = 2048


def _tc_mlp(avg2, W1d, b1d, W2d, b2d):
    def body(a_ref, w1_ref, b1_ref, w2_ref, b2_ref, o_ref):
        h = jnp.maximum(
            jnp.dot(a_ref[...], w1_ref[...], preferred_element_type=jnp.float32)
            + b1_ref[...], 0.0)
        o_ref[...] = (jnp.dot(h, w2_ref[...], preferred_element_type=jnp.float32)
                      + b2_ref[...])

    return pl.pallas_call(
        body,
        grid=(_B // 2 // _BLK,),
        in_specs=[
            pl.BlockSpec((_BLK, 2 * _EMB), lambda i: (i, 0)),
            pl.BlockSpec((2 * _EMB, 2 * _HID), lambda i: (0, 0)),
            pl.BlockSpec((1, 2 * _HID), lambda i: (0, 0)),
            pl.BlockSpec((2 * _HID, 4), lambda i: (0, 0)),
            pl.BlockSpec((1, 4), lambda i: (0, 0)),
        ],
        out_specs=pl.BlockSpec((_BLK, 4), lambda i: (i, 0)),
        out_shape=jax.ShapeDtypeStruct((_B // 2, 4), jnp.float32),
    )(avg2, W1d, b1d, W2d, b2d)


def kernel(X, lengths, emb_table, W1, b1, W2, b2):
    X = X.astype(jnp.int32)
    inv_den = 1.0 / jnp.maximum(lengths.astype(jnp.float32), 1.0)
    avg2 = _sc_embed_mean(X, _tc_pack_table(emb_table), inv_den)

    W1p = W1[_PERM, :]
    zeros1 = jnp.zeros_like(W1)
    W1d = jnp.concatenate([
        jnp.concatenate([W1p, zeros1], axis=1),
        jnp.concatenate([zeros1, W1p], axis=1)], axis=0)
    b1d = jnp.concatenate([b1, b1]).reshape(1, 2 * _HID)
    zeros2 = jnp.zeros_like(W2)
    W2d = jnp.concatenate([
        jnp.concatenate([W2, zeros2], axis=1),
        jnp.concatenate([zeros2, W2], axis=1)], axis=0)
    b2d = jnp.concatenate([b2, b2]).reshape(1, 4)

    out2 = _tc_mlp(avg2, W1d, b1d, W2d, b2d)
    return out2.reshape(_B, 2)


# speedup vs baseline: 1.4427x; 1.0697x over previous
import functools

import numpy as np

import jax
import jax.numpy as jnp
from jax import lax
from jax.experimental import pallas as pl
from jax.experimental.pallas import tpu as pltpu
from jax.experimental.pallas import tpu_sc as plsc

_B, _T = 16384, 200
_EMB = 64
_HID = 200
_NC, _NS = 2, 16
_NW = _NC * _NS
_BAGS = _B // _NW
_SPLIT = 104
_RING = 4
_UNROLL = 8
_VOCAB = 100000

_PERM = np.concatenate([
    np.arange(0, 32, 2), np.arange(1, 32, 2),
    np.arange(32, 64, 2), np.arange(33, 64, 2)])


def _sc_embed_mean(x, table, inv_den):
    mesh = plsc.VectorSubcoreMesh(core_axis_name="c", subcore_axis_name="s")

    @functools.partial(
        pl.kernel,
        mesh=mesh,
        out_type=jax.ShapeDtypeStruct((_B // 2, 2 * _EMB), jnp.float32),
        scratch_types=[
            pltpu.VMEM((_RING, _T), jnp.int32),
            pltpu.VMEM((_RING, _T, _EMB), jnp.bfloat16),
            pltpu.VMEM((_RING, _EMB), jnp.float32),
            pltpu.VMEM((_BAGS,), jnp.float32),
            pltpu.SemaphoreType.DMA,
            pltpu.SemaphoreType.DMA,
            pltpu.SemaphoreType.DMA,
        ],
        compiler_params=pltpu.CompilerParams(use_tc_tiling_on_sc=False,
                                             needs_layout_passes=False),
    )
    def body(x_hbm, tab_hbm, den_hbm, out_hbm, idx_v, rows_v, acc_v, den_v,
             isem, rsem, ssem):
        wid = lax.axis_index("s") * _NC + lax.axis_index("c")
        base = wid * _BAGS

        pltpu.sync_copy(den_hbm.at[pl.ds(base, _BAGS)], den_v)

        def fire_idx(bag, slot):
            pltpu.async_copy(x_hbm.at[pl.ds((base + bag) * _T, _T)],
                             idx_v.at[slot], isem)

        def wait_idx(slot):
            pltpu.make_async_copy(x_hbm.at[pl.ds(0, _T)],
                                  idx_v.at[slot], isem).wait()

        def fire_rows(slot):
            pltpu.async_copy(
                tab_hbm.at[idx_v.at[slot, pl.ds(0, _SPLIT)]],
                rows_v.at[slot, pl.ds(0, _SPLIT)], rsem)
            pltpu.async_copy(
                tab_hbm.at[idx_v.at[slot, pl.ds(_SPLIT, _T - _SPLIT)]],
                rows_v.at[slot, pl.ds(_SPLIT, _T - _SPLIT)], rsem)

        def wait_rows(slot):
            pltpu.make_async_copy(tab_hbm.at[pl.ds(0, _T)],
                                  rows_v.at[slot], rsem).wait()

        def out_slice(i, half):
            return out_hbm.at[(base + i) // 2, pl.ds(half * _EMB, _EMB)]

        def drain_store(slot):
            pltpu.make_async_copy(acc_v.at[slot],
                                  out_hbm.at[0, pl.ds(0, _EMB)], ssem).wait()

        for s in range(_RING):
            fire_idx(s, s)
        for s in range(2):
            wait_idx(s)
            fire_rows(s)

        def quad_body(j, carry):
            i0 = j * _RING
            for b in range(_RING):
                i = i0 + b
                @pl.when(i + 2 < _BAGS)
                def _():
                    wait_idx((b + 2) % _RING)
                    fire_rows((b + 2) % _RING)

                wait_rows(b)

                @pl.when(i + _RING < _BAGS)
                def _():
                    fire_idx(i + _RING, b)

                zero = jnp.zeros((16,), jnp.float32)

                def red(tt, accs):
                    c0, c1, c2, c3 = accs
                    for r in range(_UNROLL):
                        t = tt * _UNROLL + r
                        r0 = rows_v[b, t, pl.ds(0, 32)]
                        r1 = rows_v[b, t, pl.ds(32, 32)]
                        a0, b0 = plsc.unpack(
                            r0, format=plsc.PackFormat.INTERLEAVED,
                            preferred_element_type=jnp.float32)
                        a1, b1 = plsc.unpack(
                            r1, format=plsc.PackFormat.INTERLEAVED,
                            preferred_element_type=jnp.float32)
                        c0 = c0 + a0
                        c1 = c1 + b0
                        c2 = c2 + a1
                        c3 = c3 + b1
                    return (c0, c1, c2, c3)

                c0, c1, c2, c3 = lax.fori_loop(
                    0, _T // _UNROLL, red, (zero, zero, zero, zero))

                d = plsc.load_gather(den_v, [jnp.full((16,), i, jnp.int32)])

                @pl.when(i >= _RING)
                def _():
                    drain_store(b)

                acc_v[b, pl.ds(0, 16)] = c0 * d
                acc_v[b, pl.ds(16, 16)] = c1 * d
                acc_v[b, pl.ds(32, 16)] = c2 * d
                acc_v[b, pl.ds(48, 16)] = c3 * d
                pltpu.async_copy(acc_v.at[b], out_slice(i, b % 2), ssem)
            return carry

        lax.fori_loop(0, _BAGS // _RING, quad_body, 0)

        for s in range(_RING):
            drain_store(s)

    return body(x, table, inv_den)


_BLK = 2048


def _tc_mlp(avg2, W1d, b1d, W2d, b2d):
    def body(a_ref, w1_ref, b1_ref, w2_ref, b2_ref, o_ref):
        h = jnp.maximum(
            jnp.dot(a_ref[...], w1_ref[...], preferred_element_type=jnp.float32)
            + b1_ref[...], 0.0)
        o_ref[...] = (jnp.dot(h, w2_ref[...], preferred_element_type=jnp.float32)
                      + b2_ref[...])

    return pl.pallas_call(
        body,
        grid=(_B // 2 // _BLK,),
        in_specs=[
            pl.BlockSpec((_BLK, 2 * _EMB), lambda i: (i, 0)),
            pl.BlockSpec((2 * _EMB, 2 * _HID), lambda i: (0, 0)),
            pl.BlockSpec((1, 2 * _HID), lambda i: (0, 0)),
            pl.BlockSpec((2 * _HID, 4), lambda i: (0, 0)),
            pl.BlockSpec((1, 4), lambda i: (0, 0)),
        ],
        out_specs=pl.BlockSpec((_BLK, 4), lambda i: (i, 0)),
        out_shape=jax.ShapeDtypeStruct((_B // 2, 4), jnp.float32),
    )(avg2, W1d, b1d, W2d, b2d)


def kernel(X, lengths, emb_table, W1, b1, W2, b2):
    X = X.astype(jnp.int32).reshape(_B * _T)
    inv_den = 1.0 / jnp.maximum(lengths.astype(jnp.float32), 1.0)
    avg2 = _sc_embed_mean(X, emb_table.astype(jnp.bfloat16), inv_den)

    W1p = W1[_PERM, :]
    zeros1 = jnp.zeros_like(W1)
    W1d = jnp.concatenate([
        jnp.concatenate([W1p, zeros1], axis=1),
        jnp.concatenate([zeros1, W1p], axis=1)], axis=0)
    b1d = jnp.concatenate([b1, b1]).reshape(1, 2 * _HID)
    zeros2 = jnp.zeros_like(W2)
    W2d = jnp.concatenate([
        jnp.concatenate([W2, zeros2], axis=1),
        jnp.concatenate([zeros2, W2], axis=1)], axis=0)
    b2d = jnp.concatenate([b2, b2]).reshape(1, 4)

    out2 = _tc_mlp(avg2, W1d, b1d, W2d, b2d)
    return out2.reshape(_B, 2)
